# baseline (speedup 1.0000x reference)
import functools

import jax
import jax.numpy as jnp
from jax import lax
from jax.experimental import pallas as pl
from jax.experimental.pallas import tpu as pltpu
from jax.experimental.pallas import tpu_sc as plsc

N_ = 10000
E_ = 320000
H_ = 128
G_ = 128
C_ = 10
L_ = 3

CHUNK = 128
NCH = E_ // CHUNK
NTILES = 32
TCH = 80
SC_CH = 16 * TCH
NCH_PAD = 2 * SC_CH
ROWS_PER_TILE = 624
HTCH = TCH // 2


def _encode_body(x_ref, atom_ref, bond_ref, src_ref, e0_ref, e1_ref, e2_ref,
                 h_ref, ea8_ref, gidx_ref):
    xf = x_ref[...].astype(jnp.float32)
    a = atom_ref[...]
    diff = a[:, 1, :] - a[:, 0, :]
    base = jnp.sum(a[:, 0, :], axis=0, keepdims=True)
    h_ref[...] = lax.dot(xf, diff, preferred_element_type=jnp.float32,
                         precision=lax.Precision.HIGHEST) + base
    b = bond_ref[...]
    rows = []
    for code in range(8):
        r = b[0, code & 1] + b[1, (code >> 1) & 1] + b[2, (code >> 2) & 1]
        rows.append(r[None])
    ea8_ref[...] = jnp.concatenate(rows, axis=0)
    code = e0_ref[...] + 2 * e1_ref[...] + 4 * e2_ref[...]
    gidx_ref[...] = code * N_ + src_ref[...]


def _encode(x, atom01, bond01, src2d, e0, e1, e2):
    return pl.pallas_call(
        _encode_body,
        out_shape=[
            jax.ShapeDtypeStruct((N_, H_), jnp.float32),
            jax.ShapeDtypeStruct((8, H_), jnp.float32),
            jax.ShapeDtypeStruct((NCH, CHUNK), jnp.int32),
        ],
    )(x, atom01, bond01, src2d, e0, e1, e2)


def _expand_body(h_ref, ea_ref, r_ref):
    r_ref[...] = jnp.maximum(h_ref[...] + ea_ref[0], 0.0)[None]


def _expand(h, ea8):
    return pl.pallas_call(
        _expand_body,
        grid=(8,),
        in_specs=[
            pl.BlockSpec((N_, H_), lambda c: (0, 0)),
            pl.BlockSpec((1, 1, H_), lambda c: (c, 0, 0)),
        ],
        out_specs=pl.BlockSpec((1, N_, H_), lambda c: (c, 0, 0)),
        out_shape=jax.ShapeDtypeStruct((8, N_, H_), jnp.float32),
    )(h, ea8)


def _mlp_math(h, a0, a1, w1, b1, gamma, beta, w2, b2):
    z = h + a0 + a1
    z1 = jnp.maximum(lax.dot(z, w1, preferred_element_type=jnp.float32) + b1, 0.0)
    mu = jnp.mean(z1, axis=0, keepdims=True)
    var = jnp.mean((z1 - mu) * (z1 - mu), axis=0, keepdims=True)
    zn = (z1 - mu) / jnp.sqrt(var + 1e-5) * gamma + beta
    return lax.dot(zn, w2, preferred_element_type=jnp.float32) + b2


def _mlp_body(h_ref, a0_ref, a1_ref, w1_ref, b1_ref, g_ref, be_ref, w2_ref,
              b2_ref, o_ref):
    o_ref[...] = _mlp_math(h_ref[...], a0_ref[...], a1_ref[...], w1_ref[...],
                           b1_ref[...], g_ref[...], be_ref[...], w2_ref[...],
                           b2_ref[...])


def _mlp(h, a0, a1, w1, b1, gamma, beta, w2, b2):
    return pl.pallas_call(
        _mlp_body,
        out_shape=jax.ShapeDtypeStruct((N_, H_), jnp.float32),
    )(h, a0, a1, w1, b1, gamma, beta, w2, b2)


def _pool_body(h_ref, a0_ref, a1_ref, w1_ref, b1_ref, g_ref, be_ref, w2_ref,
               b2_ref, bt_ref, mw1_ref, mb1_ref, mw2_ref, mb2_ref, o_ref):
    h3 = _mlp_math(h_ref[...], a0_ref[...], a1_ref[...], w1_ref[...],
                   b1_ref[...], g_ref[...], be_ref[...], w2_ref[...],
                   b2_ref[...])
    gi = lax.broadcasted_iota(jnp.int32, (N_, G_), 1)
    oh = (bt_ref[...] == gi).astype(jnp.float32)
    pooled = lax.dot_general(oh, h3, (((0,), (0,)), ((), ())),
                             preferred_element_type=jnp.float32,
                             precision=lax.Precision.HIGHEST)
    t = jnp.maximum(
        lax.dot(pooled, mw1_ref[...], preferred_element_type=jnp.float32)
        + mb1_ref[...], 0.0)
    o_ref[...] = (lax.dot(t, mw2_ref[...], preferred_element_type=jnp.float32)
                  + mb2_ref[...])


def _pool(h, a0, a1, w1, b1, gamma, beta, w2, b2, bt, mw1, mb1, mw2, mb2):
    return pl.pallas_call(
        _pool_body,
        out_shape=jax.ShapeDtypeStruct((G_, C_), jnp.float32),
    )(h, a0, a1, w1, b1, gamma, beta, w2, b2, bt, mw1, mb1, mw2, mb2)


def _sc_body(r_hbm, gidx_hbm, dst_hbm, zero_hbm, out_hbm,
             idx_v, dst_v, row0, row1, aggr_sh, sem0, sem1, sem0s, sem1s):
    c = lax.axis_index("c")
    s = lax.axis_index("s")
    base = c * SC_CH + s * TCH
    rows0 = s * ROWS_PER_TILE
    pltpu.sync_copy(zero_hbm.at[pl.ds(rows0, ROWS_PER_TILE)],
                    aggr_sh.at[pl.ds(rows0, ROWS_PER_TILE)])

    @pl.when(s == 15)
    def _zero_tail():
        pltpu.sync_copy(zero_hbm.at[pl.ds(16 * ROWS_PER_TILE, 16)],
                        aggr_sh.at[pl.ds(16 * ROWS_PER_TILE, 16)])

    plsc.subcore_barrier()

    for half in range(TCH // HTCH):
        hbase = base + half * HTCH
        pltpu.sync_copy(gidx_hbm.at[pl.ds(hbase, HTCH)], idx_v)
        pltpu.sync_copy(dst_hbm.at[pl.ds(hbase, HTCH)], dst_v)
        nvalid = jnp.clip(NCH - hbase, 0, HTCH)

        @pl.when(nvalid > 0)
        def _prologue():
            pltpu.async_copy(r_hbm.at[idx_v.at[0]], row0, sem0)

        def _wait_g(row, sem):
            pltpu.make_async_copy(r_hbm.at[idx_v.at[0]], row, sem).wait()

        def _wait_s(row, sem):
            pltpu.make_async_copy(row, aggr_sh.at[dst_v.at[0]], sem).wait()

        def body(p, carry):
            j0 = 2 * p
            j1 = j0 + 1
            _wait_g(row0, sem0)

            @pl.when(j0 > 0)
            def _drain_s1():
                _wait_s(row1, sem1s)

            @pl.when(j1 < nvalid)
            def _issue_g1():
                pltpu.async_copy(r_hbm.at[idx_v.at[j1]], row1, sem1)

            pltpu.async_copy(row0, aggr_sh.at[dst_v.at[j0]], sem0s, add=True)

            @pl.when(j1 < nvalid)
            def _second():
                _wait_g(row1, sem1)
                _wait_s(row0, sem0s)

                @pl.when(j0 + 2 < nvalid)
                def _issue_g0():
                    pltpu.async_copy(r_hbm.at[idx_v.at[j0 + 2]], row0, sem0)

                pltpu.async_copy(row1, aggr_sh.at[dst_v.at[j1]], sem1s, add=True)

            return carry

        lax.fori_loop(0, (nvalid + 1) // 2, body, 0)

        @pl.when((nvalid > 0) & (nvalid % 2 == 1))
        def _drain_even():
            _wait_s(row0, sem0s)

        @pl.when((nvalid > 0) & (nvalid % 2 == 0))
        def _drain_odd():
            _wait_s(row1, sem1s)
    plsc.subcore_barrier()
    pltpu.sync_copy(aggr_sh.at[pl.ds(rows0, ROWS_PER_TILE)],
                    out_hbm.at[c, pl.ds(rows0, ROWS_PER_TILE)])

    @pl.when(s == 15)
    def _write_tail():
        pltpu.sync_copy(aggr_sh.at[pl.ds(16 * ROWS_PER_TILE, 16)],
                        out_hbm.at[c, pl.ds(16 * ROWS_PER_TILE, 16)])


_sc_aggregate = functools.partial(
    pl.kernel,
    out_type=jax.ShapeDtypeStruct((2, N_, H_), jnp.float32),
    mesh=plsc.VectorSubcoreMesh(core_axis_name="c", subcore_axis_name="s"),
    scratch_types=[
        pltpu.VMEM((HTCH, CHUNK), jnp.int32),
        pltpu.VMEM((HTCH, CHUNK), jnp.int32),
        pltpu.VMEM((CHUNK, H_), jnp.float32),
        pltpu.VMEM((CHUNK, H_), jnp.float32),
        pltpu.VMEM_SHARED((N_, H_), jnp.float32),
        pltpu.SemaphoreType.DMA,
        pltpu.SemaphoreType.DMA,
        pltpu.SemaphoreType.DMA,
        pltpu.SemaphoreType.DMA,
    ],
)(_sc_body)


def kernel(x, edge_index, edge_attr, batch, params):
    atom01 = jnp.stack([t[0:2] for t in params["atom_emb"]])
    bond01 = jnp.stack([t[0:2] for t in params["bond_emb"]])
    src2d = edge_index[0].reshape(NCH, CHUNK)
    e0 = edge_attr[:, 0].reshape(NCH, CHUNK)
    e1 = edge_attr[:, 1].reshape(NCH, CHUNK)
    e2 = edge_attr[:, 2].reshape(NCH, CHUNK)

    h, ea8, gidx = _encode(x, atom01, bond01, src2d, e0, e1, e2)

    pad = ((0, NCH_PAD - NCH), (0, 0))
    gidxp = jnp.pad(gidx, pad)
    dstp = jnp.pad(edge_index[1].reshape(NCH, CHUNK), pad)
    zeros = jnp.zeros((N_, H_), jnp.float32)
    bt = batch.reshape(N_, 1)

    out = None
    for l in range(L_):
        p = params["layers"][l]
        r = _expand(h, ea8.reshape(8, 1, H_)).reshape(8 * N_, H_)
        agg = _sc_aggregate(r, gidxp, dstp, zeros)
        args = (h, agg[0], agg[1], p["W1"], p["b1"].reshape(1, H_),
                p["gamma"].reshape(1, H_), p["beta"].reshape(1, H_),
                p["W2"], p["b2"].reshape(1, H_))
        if l < L_ - 1:
            h = _mlp(*args)
        else:
            mp = params["mlp"]
            out = _pool(*args, bt, mp["W1"], mp["b1"].reshape(1, H_),
                        mp["W2"], mp["b2"].reshape(1, C_))
    return out


# speedup vs baseline: 1.0199x; 1.0199x over previous
import functools

import jax
import jax.numpy as jnp
from jax import lax
from jax.experimental import pallas as pl
from jax.experimental.pallas import tpu as pltpu
from jax.experimental.pallas import tpu_sc as plsc

N_ = 10000
E_ = 320000
H_ = 128
G_ = 128
C_ = 10
L_ = 3

CHUNK = 128
NCH = E_ // CHUNK
CH64 = 64
NCH64 = E_ // CH64
TCH64 = 160
NCH64_PAD = 32 * TCH64
STG = 40
ROWS_PER_TILE = 624


def _encode_body(x_ref, atom_ref, bond_ref, src_ref, e0_ref, e1_ref, e2_ref,
                 h_ref, ea8_ref, gidx_ref):
    xf = x_ref[...].astype(jnp.float32)
    a = atom_ref[...]
    diff = a[:, 1, :] - a[:, 0, :]
    base = jnp.sum(a[:, 0, :], axis=0, keepdims=True)
    h_ref[...] = lax.dot(xf, diff, preferred_element_type=jnp.float32,
                         precision=lax.Precision.HIGHEST) + base
    b = bond_ref[...]
    rows = []
    for code in range(8):
        r = b[0, code & 1] + b[1, (code >> 1) & 1] + b[2, (code >> 2) & 1]
        rows.append(r[None])
    ea8_ref[...] = jnp.concatenate(rows, axis=0)
    code = e0_ref[...] + 2 * e1_ref[...] + 4 * e2_ref[...]
    gidx_ref[...] = code * N_ + src_ref[...]


def _encode(x, atom01, bond01, src2d, e0, e1, e2):
    return pl.pallas_call(
        _encode_body,
        out_shape=[
            jax.ShapeDtypeStruct((N_, H_), jnp.float32),
            jax.ShapeDtypeStruct((8, H_), jnp.float32),
            jax.ShapeDtypeStruct((NCH, CHUNK), jnp.int32),
        ],
    )(x, atom01, bond01, src2d, e0, e1, e2)


def _expand_body(h_ref, ea_ref, r_ref):
    r_ref[...] = jnp.maximum(h_ref[...] + ea_ref[0], 0.0)[None]


def _expand(h, ea8):
    return pl.pallas_call(
        _expand_body,
        grid=(8,),
        in_specs=[
            pl.BlockSpec((N_, H_), lambda c: (0, 0)),
            pl.BlockSpec((1, 1, H_), lambda c: (c, 0, 0)),
        ],
        out_specs=pl.BlockSpec((1, N_, H_), lambda c: (c, 0, 0)),
        out_shape=jax.ShapeDtypeStruct((8, N_, H_), jnp.float32),
    )(h, ea8)


def _mlp_math(h, a0, a1, w1, b1, gamma, beta, w2, b2):
    z = h + a0 + a1
    z1 = jnp.maximum(lax.dot(z, w1, preferred_element_type=jnp.float32) + b1, 0.0)
    mu = jnp.mean(z1, axis=0, keepdims=True)
    var = jnp.mean((z1 - mu) * (z1 - mu), axis=0, keepdims=True)
    zn = (z1 - mu) / jnp.sqrt(var + 1e-5) * gamma + beta
    return lax.dot(zn, w2, preferred_element_type=jnp.float32) + b2


def _mlp_body(h_ref, a0_ref, a1_ref, w1_ref, b1_ref, g_ref, be_ref, w2_ref,
              b2_ref, o_ref):
    o_ref[...] = _mlp_math(h_ref[...], a0_ref[...], a1_ref[...], w1_ref[...],
                           b1_ref[...], g_ref[...], be_ref[...], w2_ref[...],
                           b2_ref[...])


def _mlp(h, a0, a1, w1, b1, gamma, beta, w2, b2):
    return pl.pallas_call(
        _mlp_body,
        out_shape=jax.ShapeDtypeStruct((N_, H_), jnp.float32),
    )(h, a0, a1, w1, b1, gamma, beta, w2, b2)


def _pool_body(h_ref, a0_ref, a1_ref, w1_ref, b1_ref, g_ref, be_ref, w2_ref,
               b2_ref, bt_ref, mw1_ref, mb1_ref, mw2_ref, mb2_ref, o_ref):
    h3 = _mlp_math(h_ref[...], a0_ref[...], a1_ref[...], w1_ref[...],
                   b1_ref[...], g_ref[...], be_ref[...], w2_ref[...],
                   b2_ref[...])
    gi = lax.broadcasted_iota(jnp.int32, (N_, G_), 1)
    oh = (bt_ref[...] == gi).astype(jnp.float32)
    pooled = lax.dot_general(oh, h3, (((0,), (0,)), ((), ())),
                             preferred_element_type=jnp.float32,
                             precision=lax.Precision.HIGHEST)
    t = jnp.maximum(
        lax.dot(pooled, mw1_ref[...], preferred_element_type=jnp.float32)
        + mb1_ref[...], 0.0)
    o_ref[...] = (lax.dot(t, mw2_ref[...], preferred_element_type=jnp.float32)
                  + mb2_ref[...])


def _pool(h, a0, a1, w1, b1, gamma, beta, w2, b2, bt, mw1, mb1, mw2, mb2):
    return pl.pallas_call(
        _pool_body,
        out_shape=jax.ShapeDtypeStruct((G_, C_), jnp.float32),
    )(h, a0, a1, w1, b1, gamma, beta, w2, b2, bt, mw1, mb1, mw2, mb2)


def _sc_body(r_hbm, gidx_hbm, dst_hbm, zero_hbm, out_hbm,
             idx_v, dst_v, row0, row1, row2, row3,
             aggr_sh, g0, g1, g2, g3, s0, s1, s2, s3):
    c = lax.axis_index("c")
    s = lax.axis_index("s")
    base = (c * 16 + s) * TCH64
    rows0 = s * ROWS_PER_TILE
    rows_ring = [row0, row1, row2, row3]
    gsems = [g0, g1, g2, g3]
    ssems = [s0, s1, s2, s3]

    pltpu.sync_copy(zero_hbm.at[pl.ds(rows0, ROWS_PER_TILE)],
                    aggr_sh.at[pl.ds(rows0, ROWS_PER_TILE)])

    @pl.when(s == 15)
    def _zero_tail():
        pltpu.sync_copy(zero_hbm.at[pl.ds(16 * ROWS_PER_TILE, 16)],
                        aggr_sh.at[pl.ds(16 * ROWS_PER_TILE, 16)])

    plsc.subcore_barrier()

    def issue_g(jj, k):
        pltpu.async_copy(r_hbm.at[idx_v.at[jj]], rows_ring[k], gsems[k])

    def wait_g(k):
        pltpu.make_async_copy(r_hbm.at[idx_v.at[0]], rows_ring[k],
                              gsems[k]).wait()

    def issue_s(jj, k):
        pltpu.async_copy(rows_ring[k], aggr_sh.at[dst_v.at[jj]], ssems[k],
                         add=True)

    def wait_s(k):
        pltpu.make_async_copy(rows_ring[k], aggr_sh.at[dst_v.at[0]],
                              ssems[k]).wait()

    for stg in range(TCH64 // STG):
        sbase = base + stg * STG
        pltpu.sync_copy(gidx_hbm.at[pl.ds(sbase, STG)], idx_v)
        pltpu.sync_copy(dst_hbm.at[pl.ds(sbase, STG)], dst_v)
        nv = jnp.clip(NCH64 - sbase, 0, STG)

        for k in range(2):
            @pl.when(k < nv)
            def _prologue(k=k):
                issue_g(k, k)

        def body(m, carry):
            for k in range(4):
                jj = 4 * m + k

                @pl.when(jj < nv)
                def _consume(jj=jj, k=k):
                    wait_g(k)
                    issue_s(jj, k)

                @pl.when((jj >= 2) & (jj - 2 < nv))
                def _drain(k=k):
                    wait_s((k + 2) % 4)

                @pl.when(jj + 2 < nv)
                def _prefetch(jj=jj, k=k):
                    issue_g(jj + 2, (k + 2) % 4)

            return carry

        lax.fori_loop(0, (nv + 2 + 3) // 4, body, 0)

    plsc.subcore_barrier()
    pltpu.sync_copy(aggr_sh.at[pl.ds(rows0, ROWS_PER_TILE)],
                    out_hbm.at[c, pl.ds(rows0, ROWS_PER_TILE)])

    @pl.when(s == 15)
    def _write_tail():
        pltpu.sync_copy(aggr_sh.at[pl.ds(16 * ROWS_PER_TILE, 16)],
                        out_hbm.at[c, pl.ds(16 * ROWS_PER_TILE, 16)])


_sc_aggregate = functools.partial(
    pl.kernel,
    out_type=jax.ShapeDtypeStruct((2, N_, H_), jnp.float32),
    mesh=plsc.VectorSubcoreMesh(core_axis_name="c", subcore_axis_name="s"),
    scratch_types=(
        [pltpu.VMEM((STG, CH64), jnp.int32),
         pltpu.VMEM((STG, CH64), jnp.int32)]
        + [pltpu.VMEM((CH64, H_), jnp.float32)] * 4
        + [pltpu.VMEM_SHARED((N_, H_), jnp.float32)]
        + [pltpu.SemaphoreType.DMA] * 8
    ),
)(_sc_body)


def kernel(x, edge_index, edge_attr, batch, params):
    atom01 = jnp.stack([t[0:2] for t in params["atom_emb"]])
    bond01 = jnp.stack([t[0:2] for t in params["bond_emb"]])
    src2d = edge_index[0].reshape(NCH, CHUNK)
    e0 = edge_attr[:, 0].reshape(NCH, CHUNK)
    e1 = edge_attr[:, 1].reshape(NCH, CHUNK)
    e2 = edge_attr[:, 2].reshape(NCH, CHUNK)

    h, ea8, gidx = _encode(x, atom01, bond01, src2d, e0, e1, e2)

    pad = ((0, NCH64_PAD - NCH64), (0, 0))
    gidxp = jnp.pad(gidx.reshape(NCH64, CH64), pad)
    dstp = jnp.pad(edge_index[1].reshape(NCH64, CH64), pad)
    zeros = jnp.zeros((N_, H_), jnp.float32)
    bt = batch.reshape(N_, 1)

    out = None
    for l in range(L_):
        p = params["layers"][l]
        r = _expand(h, ea8.reshape(8, 1, H_)).reshape(8 * N_, H_)
        agg = _sc_aggregate(r, gidxp, dstp, zeros)
        args = (h, agg[0], agg[1], p["W1"], p["b1"].reshape(1, H_),
                p["gamma"].reshape(1, H_), p["beta"].reshape(1, H_),
                p["W2"], p["b2"].reshape(1, H_))
        if l < L_ - 1:
            h = _mlp(*args)
        else:
            mp = params["mlp"]
            out = _pool(*args, bt, mp["W1"], mp["b1"].reshape(1, H_),
                        mp["W2"], mp["b2"].reshape(1, C_))
    return out


# speedup vs baseline: 1.1422x; 1.1200x over previous
import functools

import jax
import jax.numpy as jnp
from jax import lax
from jax.experimental import pallas as pl
from jax.experimental.pallas import tpu as pltpu
from jax.experimental.pallas import tpu_sc as plsc

N_ = 10000
E_ = 320000
H_ = 128
G_ = 128
C_ = 10
L_ = 3

CHUNK = 128
NCH = E_ // CHUNK
TCH = 80
SC_CH = 16 * TCH
NCH_PAD = 2 * SC_CH
ROWS_PER_TILE = 624
HTCH = TCH // 2


def _encode_body(x_ref, atom_ref, bond_ref, src_ref, e0_ref, e1_ref, e2_ref,
                 h_ref, ea8_ref, gidx_ref, r_ref):
    cstep = pl.program_id(0)

    @pl.when(cstep == 0)
    def _first():
        xf = x_ref[...].astype(jnp.float32)
        a = atom_ref[...]
        diff = a[:, 1, :] - a[:, 0, :]
        base = jnp.sum(a[:, 0, :], axis=0, keepdims=True)
        h_ref[...] = lax.dot(xf, diff, preferred_element_type=jnp.float32,
                             precision=lax.Precision.HIGHEST) + base
        b = bond_ref[...]
        rows = []
        for code in range(8):
            r = b[0, code & 1] + b[1, (code >> 1) & 1] + b[2, (code >> 2) & 1]
            rows.append(r[None])
        ea8_ref[...] = jnp.concatenate(rows, axis=0)
        code = e0_ref[...] + 2 * e1_ref[...] + 4 * e2_ref[...]
        gidx_ref[...] = code * N_ + src_ref[...]

    sel = (lax.broadcasted_iota(jnp.int32, (8, 1), 0) == cstep
           ).astype(jnp.float32)
    row = jnp.sum(ea8_ref[...] * sel, axis=0, keepdims=True)
    r_ref[...] = jnp.maximum(h_ref[...] + row, 0.0)[None]


def _encode(x, atom01, bond01, src2d, e0, e1, e2):
    z2 = lambda c: (0, 0)
    z3 = lambda c: (0, 0, 0)
    return pl.pallas_call(
        _encode_body,
        grid=(8,),
        in_specs=[
            pl.BlockSpec((N_, 9), z2),
            pl.BlockSpec((9, 2, H_), z3),
            pl.BlockSpec((3, 2, H_), z3),
            pl.BlockSpec((NCH, CHUNK), z2),
            pl.BlockSpec((NCH, CHUNK), z2),
            pl.BlockSpec((NCH, CHUNK), z2),
            pl.BlockSpec((NCH, CHUNK), z2),
        ],
        out_specs=[
            pl.BlockSpec((N_, H_), z2),
            pl.BlockSpec((8, H_), z2),
            pl.BlockSpec((NCH, CHUNK), z2),
            pl.BlockSpec((1, N_, H_), lambda c: (c, 0, 0)),
        ],
        out_shape=[
            jax.ShapeDtypeStruct((N_, H_), jnp.float32),
            jax.ShapeDtypeStruct((8, H_), jnp.float32),
            jax.ShapeDtypeStruct((NCH, CHUNK), jnp.int32),
            jax.ShapeDtypeStruct((8, N_, H_), jnp.float32),
        ],
    )(x, atom01, bond01, src2d, e0, e1, e2)


def _mlp_math(h, a0, a1, w1, b1, gamma, beta, w2, b2):
    z = h + a0 + a1
    z1 = jnp.maximum(lax.dot(z, w1, preferred_element_type=jnp.float32) + b1, 0.0)
    mu = jnp.mean(z1, axis=0, keepdims=True)
    var = jnp.mean((z1 - mu) * (z1 - mu), axis=0, keepdims=True)
    zn = (z1 - mu) / jnp.sqrt(var + 1e-5) * gamma + beta
    return lax.dot(zn, w2, preferred_element_type=jnp.float32) + b2


def _mlp_body(h_ref, a0_ref, a1_ref, w1_ref, b1_ref, g_ref, be_ref, w2_ref,
              b2_ref, ea8_ref, o_ref, r_ref):
    cstep = pl.program_id(0)

    @pl.when(cstep == 0)
    def _first():
        o_ref[...] = _mlp_math(h_ref[...], a0_ref[...], a1_ref[...],
                               w1_ref[...], b1_ref[...], g_ref[...],
                               be_ref[...], w2_ref[...], b2_ref[...])

    sel = (lax.broadcasted_iota(jnp.int32, (8, 1), 0) == cstep
           ).astype(jnp.float32)
    row = jnp.sum(ea8_ref[...] * sel, axis=0, keepdims=True)
    r_ref[...] = jnp.maximum(o_ref[...] + row, 0.0)[None]


def _mlp(h, a0, a1, w1, b1, gamma, beta, w2, b2, ea8):
    z2 = lambda c: (0, 0)
    nh = pl.BlockSpec((N_, H_), z2)
    row1 = pl.BlockSpec((1, H_), z2)
    return pl.pallas_call(
        _mlp_body,
        grid=(8,),
        in_specs=[nh, nh, nh,
                  pl.BlockSpec((H_, H_), z2), row1, row1, row1,
                  pl.BlockSpec((H_, H_), z2), row1,
                  pl.BlockSpec((8, H_), z2)],
        out_specs=[nh, pl.BlockSpec((1, N_, H_), lambda c: (c, 0, 0))],
        out_shape=[jax.ShapeDtypeStruct((N_, H_), jnp.float32),
                   jax.ShapeDtypeStruct((8, N_, H_), jnp.float32)],
    )(h, a0, a1, w1, b1, gamma, beta, w2, b2, ea8)


def _pool_body(h_ref, a0_ref, a1_ref, w1_ref, b1_ref, g_ref, be_ref, w2_ref,
               b2_ref, bt_ref, mw1_ref, mb1_ref, mw2_ref, mb2_ref, o_ref):
    h3 = _mlp_math(h_ref[...], a0_ref[...], a1_ref[...], w1_ref[...],
                   b1_ref[...], g_ref[...], be_ref[...], w2_ref[...],
                   b2_ref[...])
    gi = lax.broadcasted_iota(jnp.int32, (N_, G_), 1)
    oh = (bt_ref[...] == gi).astype(jnp.float32)
    pooled = lax.dot_general(oh, h3, (((0,), (0,)), ((), ())),
                             preferred_element_type=jnp.float32,
                             precision=lax.Precision.HIGHEST)
    t = jnp.maximum(
        lax.dot(pooled, mw1_ref[...], preferred_element_type=jnp.float32)
        + mb1_ref[...], 0.0)
    o_ref[...] = (lax.dot(t, mw2_ref[...], preferred_element_type=jnp.float32)
                  + mb2_ref[...])


def _pool(h, a0, a1, w1, b1, gamma, beta, w2, b2, bt, mw1, mb1, mw2, mb2):
    return pl.pallas_call(
        _pool_body,
        out_shape=jax.ShapeDtypeStruct((G_, C_), jnp.float32),
    )(h, a0, a1, w1, b1, gamma, beta, w2, b2, bt, mw1, mb1, mw2, mb2)


def _sc_body(r_hbm, gidx_hbm, dst_hbm, zero_hbm, out_hbm,
             idx_v, dst_v, row0, row1, aggr_sh, sem0, sem1):
    c = lax.axis_index("c")
    s = lax.axis_index("s")
    base = c * SC_CH + s * TCH
    rows0 = s * ROWS_PER_TILE
    pltpu.sync_copy(zero_hbm.at[pl.ds(rows0, ROWS_PER_TILE)],
                    aggr_sh.at[pl.ds(rows0, ROWS_PER_TILE)])

    @pl.when(s == 15)
    def _zero_tail():
        pltpu.sync_copy(zero_hbm.at[pl.ds(16 * ROWS_PER_TILE, 16)],
                        aggr_sh.at[pl.ds(16 * ROWS_PER_TILE, 16)])

    plsc.subcore_barrier()

    for half in range(TCH // HTCH):
        hbase = base + half * HTCH
        pltpu.sync_copy(gidx_hbm.at[pl.ds(hbase, HTCH)], idx_v)
        pltpu.sync_copy(dst_hbm.at[pl.ds(hbase, HTCH)], dst_v)
        nvalid = jnp.clip(NCH - hbase, 0, HTCH)

        @pl.when(nvalid > 0)
        def _prologue():
            pltpu.async_copy(r_hbm.at[idx_v.at[0]], row0, sem0)

        def body(p, carry):
            j0 = 2 * p
            j1 = j0 + 1

            @pl.when(j1 < nvalid)
            def _issue1():
                pltpu.async_copy(r_hbm.at[idx_v.at[j1]], row1, sem1)

            pltpu.make_async_copy(r_hbm.at[idx_v.at[j0]], row0, sem0).wait()
            pltpu.sync_copy(row0, aggr_sh.at[dst_v.at[j0]], add=True)

            @pl.when(j0 + 2 < nvalid)
            def _issue0():
                pltpu.async_copy(r_hbm.at[idx_v.at[j0 + 2]], row0, sem0)

            @pl.when(j1 < nvalid)
            def _drain1():
                pltpu.make_async_copy(r_hbm.at[idx_v.at[j1]], row1, sem1).wait()
                pltpu.sync_copy(row1, aggr_sh.at[dst_v.at[j1]], add=True)

            return carry

        lax.fori_loop(0, (nvalid + 1) // 2, body, 0)

    plsc.subcore_barrier()
    pltpu.sync_copy(aggr_sh.at[pl.ds(rows0, ROWS_PER_TILE)],
                    out_hbm.at[c, pl.ds(rows0, ROWS_PER_TILE)])

    @pl.when(s == 15)
    def _write_tail():
        pltpu.sync_copy(aggr_sh.at[pl.ds(16 * ROWS_PER_TILE, 16)],
                        out_hbm.at[c, pl.ds(16 * ROWS_PER_TILE, 16)])


_sc_aggregate = functools.partial(
    pl.kernel,
    out_type=jax.ShapeDtypeStruct((2, N_, H_), jnp.float32),
    mesh=plsc.VectorSubcoreMesh(core_axis_name="c", subcore_axis_name="s"),
    scratch_types=[
        pltpu.VMEM((HTCH, CHUNK), jnp.int32),
        pltpu.VMEM((HTCH, CHUNK), jnp.int32),
        pltpu.VMEM((CHUNK, H_), jnp.float32),
        pltpu.VMEM((CHUNK, H_), jnp.float32),
        pltpu.VMEM_SHARED((N_, H_), jnp.float32),
        pltpu.SemaphoreType.DMA,
        pltpu.SemaphoreType.DMA,
    ],
)(_sc_body)


def kernel(x, edge_index, edge_attr, batch, params):
    atom01 = jnp.stack([t[0:2] for t in params["atom_emb"]])
    bond01 = jnp.stack([t[0:2] for t in params["bond_emb"]])
    src2d = edge_index[0].reshape(NCH, CHUNK)
    e0 = edge_attr[:, 0].reshape(NCH, CHUNK)
    e1 = edge_attr[:, 1].reshape(NCH, CHUNK)
    e2 = edge_attr[:, 2].reshape(NCH, CHUNK)

    h, ea8, gidx, r = _encode(x, atom01, bond01, src2d, e0, e1, e2)

    pad = ((0, NCH_PAD - NCH), (0, 0))
    gidxp = jnp.pad(gidx, pad)
    dstp = jnp.pad(edge_index[1].reshape(NCH, CHUNK), pad)
    zeros = jnp.zeros((N_, H_), jnp.float32)
    bt = batch.reshape(N_, 1)

    out = None
    for l in range(L_):
        p = params["layers"][l]
        agg = _sc_aggregate(r.reshape(8 * N_, H_), gidxp, dstp, zeros)
        args = (h, agg[0], agg[1], p["W1"], p["b1"].reshape(1, H_),
                p["gamma"].reshape(1, H_), p["beta"].reshape(1, H_),
                p["W2"], p["b2"].reshape(1, H_))
        if l < L_ - 1:
            h, r = _mlp(*args, ea8)
        else:
            mp = params["mlp"]
            out = _pool(*args, bt, mp["W1"], mp["b1"].reshape(1, H_),
                        mp["W2"], mp["b2"].reshape(1, C_))
    return out
